# Initial kernel scaffold; baseline (speedup 1.0000x reference)
import jax
import jax.numpy as jnp
from jax.experimental import pallas as pl


def kernel(x, edge_index, edge_attr, We1, be1, We2, be2, Wn1, bn1, Wn2, bn2):
    raise NotImplementedError("write your pallas kernel here")


# speedup vs baseline: 2.7297x; 2.7297x over previous
import functools

import jax
import jax.numpy as jnp
from jax import lax
from jax.experimental import pallas as pl
from jax.experimental.pallas import tpu as pltpu
from jax.experimental.pallas import tpu_sc as plsc

N = 10000
E = 320000
D = 128
DE = 16
H = 64

NC = 2
NS = 16
NW = NC * NS
CH_E = 128

NCHUNK = 80
EW = NCHUNK * CH_E
EP = EW * NW

RPT = ((N + NS - 1) // NS + 7) // 8 * 8
NP = RPT * NS


def _proj_body(x_ref, wa_ref, wb_ref, xr_ref, xs_ref):
    xb = x_ref[...]
    xr_ref[...] = jnp.dot(xb, wa_ref[...], preferred_element_type=jnp.float32)
    xs_ref[...] = jnp.dot(xb, wb_ref[...], preferred_element_type=jnp.float32)


def _project(x, wa, wb):
    blk = 1000
    grid = N // blk
    return pl.pallas_call(
        _proj_body,
        grid=(grid,),
        in_specs=[
            pl.BlockSpec((blk, D), lambda i: (i, 0)),
            pl.BlockSpec((D, H), lambda i: (0, 0)),
            pl.BlockSpec((D, H), lambda i: (0, 0)),
        ],
        out_specs=[
            pl.BlockSpec((blk, H), lambda i: (i, 0)),
            pl.BlockSpec((blk, H), lambda i: (i, 0)),
        ],
        out_shape=[
            jax.ShapeDtypeStruct((N, H), jnp.float32),
            jax.ShapeDtypeStruct((N, H), jnp.float32),
        ],
    )(x, wa, wb)


def _gather_body(xr_hbm, xs_hbm, idxr_hbm, idxs_hbm, g_hbm,
                 idxr_v, idxs_v, bufr, bufs, semr, sems):
    c = lax.axis_index("c")
    s = lax.axis_index("s")
    wid = s * NC + c
    ibase = pl.multiple_of(wid * NCHUNK, 8)
    pltpu.sync_copy(idxr_hbm.at[pl.ds(ibase, NCHUNK)], idxr_v)
    pltpu.sync_copy(idxs_hbm.at[pl.ds(ibase, NCHUNK)], idxs_v)
    ebase = wid * EW

    def chunk(j, carry):
        cpr = pltpu.async_copy(xr_hbm.at[idxr_v.at[j]], bufr, semr)
        cps = pltpu.async_copy(xs_hbm.at[idxs_v.at[j]], bufs, sems)
        cpr.wait()
        cps.wait()

        def row(i, c2):
            for cc in range(H // 16):
                sl = pl.ds(cc * 16, 16)
                bufr[i, sl] = bufr[i, sl] + bufs[i, sl]
            return c2

        lax.fori_loop(0, CH_E, row, 0)
        off = pl.multiple_of(ebase + j * CH_E, 8)
        pltpu.sync_copy(bufr, g_hbm.at[pl.ds(off, CH_E)])
        return carry

    lax.fori_loop(0, NCHUNK, chunk, 0)


def _edge_gather(xr, xs, idxr2d, idxs2d):
    mesh = plsc.VectorSubcoreMesh(core_axis_name="c", subcore_axis_name="s")
    f = pl.kernel(
        _gather_body,
        out_type=jax.ShapeDtypeStruct((EP, H), jnp.float32),
        mesh=mesh,
        compiler_params=pltpu.CompilerParams(use_tc_tiling_on_sc=False),
        scratch_types=[
            pltpu.VMEM((NCHUNK, CH_E), jnp.int32),
            pltpu.VMEM((NCHUNK, CH_E), jnp.int32),
            pltpu.VMEM((CH_E, H), jnp.float32),
            pltpu.VMEM((CH_E, H), jnp.float32),
            pltpu.SemaphoreType.DMA,
            pltpu.SemaphoreType.DMA,
        ],
    )
    return f(xr, xs, idxr2d, idxs2d)


def _edge_mlp_body(g_ref, ea_ref, w1c_ref, be1_ref, w2_ref, be2_ref, out_ref):
    ea = ea_ref[...]
    h = g_ref[...] + jnp.dot(ea, w1c_ref[...], preferred_element_type=jnp.float32)
    h = jnp.maximum(h + be1_ref[...], 0.0)
    out_ref[...] = ea + jnp.dot(h, w2_ref[...], preferred_element_type=jnp.float32) + be2_ref[...]


def _edge_mlp(g, ea_pad, w1c, be1, w2, be2):
    blk = 2048
    grid = EP // blk
    return pl.pallas_call(
        _edge_mlp_body,
        grid=(grid,),
        in_specs=[
            pl.BlockSpec((blk, H), lambda i: (i, 0)),
            pl.BlockSpec((blk, DE), lambda i: (i, 0)),
            pl.BlockSpec((DE, H), lambda i: (0, 0)),
            pl.BlockSpec((1, H), lambda i: (0, 0)),
            pl.BlockSpec((H, DE), lambda i: (0, 0)),
            pl.BlockSpec((1, DE), lambda i: (0, 0)),
        ],
        out_specs=pl.BlockSpec((blk, DE), lambda i: (i, 0)),
        out_shape=jax.ShapeDtypeStruct((EP, DE), jnp.float32),
    )(g, ea_pad, w1c, be1, w2, be2)


def _scatter_body(edges_hbm, sidx_hbm, out_hbm, sidx_v, ebuf, zbuf, acc, sem):
    c = lax.axis_index("c")
    s = lax.axis_index("s")
    wid = s * NC + c

    def zrow(i, carry):
        zbuf[i, :] = jnp.zeros((16,), jnp.float32)
        return carry

    lax.fori_loop(0, RPT, zrow, 0)

    rbase = pl.multiple_of(s * RPT, 8)
    pltpu.sync_copy(zbuf, acc.at[pl.ds(rbase, RPT)])
    plsc.subcore_barrier()
    pltpu.sync_copy(sidx_hbm.at[pl.ds(pl.multiple_of(wid * NCHUNK, 8), NCHUNK)], sidx_v)
    ebase = wid * EW

    def chunk(j, carry):
        off = pl.multiple_of(ebase + j * CH_E, 8)
        pltpu.sync_copy(edges_hbm.at[pl.ds(off, CH_E)], ebuf)
        pltpu.sync_copy(ebuf, acc.at[sidx_v.at[j]], add=True)
        return carry

    lax.fori_loop(0, NCHUNK, chunk, 0)
    plsc.subcore_barrier()
    pltpu.sync_copy(acc.at[pl.ds(rbase, RPT)], zbuf)
    pltpu.sync_copy(zbuf, out_hbm.at[c, pl.ds(rbase, RPT)])


def _segment_scatter(edges_new, sidx2d):
    mesh = plsc.VectorSubcoreMesh(core_axis_name="c", subcore_axis_name="s")
    f = pl.kernel(
        _scatter_body,
        out_type=jax.ShapeDtypeStruct((NC, NP, DE), jnp.float32),
        mesh=mesh,
        compiler_params=pltpu.CompilerParams(use_tc_tiling_on_sc=False),
        scratch_types=[
            pltpu.VMEM((NCHUNK, CH_E), jnp.int32),
            pltpu.VMEM((CH_E, DE), jnp.float32),
            pltpu.VMEM((RPT, DE), jnp.float32),
            pltpu.VMEM_SHARED((NP, DE), jnp.float32),
            pltpu.SemaphoreType.DMA,
        ],
    )
    return f(edges_new, sidx2d)


def _node_body(x_ref, p_ref, wa_ref, wb_ref, bn1_ref, w2_ref, bn2_ref, out_ref):
    xb = x_ref[...]
    a = p_ref[0] + p_ref[1]
    h = jnp.dot(xb, wa_ref[...], preferred_element_type=jnp.float32)
    h = h + jnp.dot(a, wb_ref[...], preferred_element_type=jnp.float32)
    h = jnp.maximum(h + bn1_ref[...], 0.0)
    out_ref[...] = xb + jnp.dot(h, w2_ref[...], preferred_element_type=jnp.float32) + bn2_ref[...]


def _node_mlp(x, partials, wa, wb, bn1, w2, bn2):
    blk = 1000
    grid = N // blk
    return pl.pallas_call(
        _node_body,
        grid=(grid,),
        in_specs=[
            pl.BlockSpec((blk, D), lambda i: (i, 0)),
            pl.BlockSpec((NC, blk, DE), lambda i: (0, i, 0)),
            pl.BlockSpec((D, H), lambda i: (0, 0)),
            pl.BlockSpec((DE, H), lambda i: (0, 0)),
            pl.BlockSpec((1, H), lambda i: (0, 0)),
            pl.BlockSpec((H, D), lambda i: (0, 0)),
            pl.BlockSpec((1, D), lambda i: (0, 0)),
        ],
        out_specs=pl.BlockSpec((blk, D), lambda i: (i, 0)),
        out_shape=jax.ShapeDtypeStruct((N, D), jnp.float32),
    )(x, partials, wa, wb, bn1, w2, bn2)


def kernel(x, edge_index, edge_attr, We1, be1, We2, be2, Wn1, bn1, Wn2, bn2):
    senders = edge_index[0].astype(jnp.int32)
    receivers = edge_index[1].astype(jnp.int32)
    pad = EP - E
    gr = jnp.concatenate([receivers, jnp.zeros((pad,), jnp.int32)]).reshape(-1, CH_E)
    gs = jnp.concatenate([senders, jnp.zeros((pad,), jnp.int32)]).reshape(-1, CH_E)
    sr = jnp.concatenate([receivers, jnp.full((pad,), N, jnp.int32)]).reshape(-1, CH_E)
    ea_pad = jnp.concatenate([edge_attr, jnp.zeros((pad, DE), jnp.float32)])

    xr, xs = _project(x, We1[:D], We1[D:2 * D])
    g = _edge_gather(xr, xs, gr, gs)
    edges_new = _edge_mlp(g, ea_pad, We1[2 * D:], be1.reshape(1, H),
                          We2, be2.reshape(1, DE))
    partials = _segment_scatter(edges_new, sr)
    nodes_new = _node_mlp(x, partials, Wn1[:D], Wn1[D:], bn1.reshape(1, H),
                          Wn2, bn2.reshape(1, D))
    return nodes_new


# speedup vs baseline: 2.9405x; 1.0772x over previous
import functools

import jax
import jax.numpy as jnp
from jax import lax
from jax.experimental import pallas as pl
from jax.experimental.pallas import tpu as pltpu
from jax.experimental.pallas import tpu_sc as plsc

N = 10000
E = 320000
D = 128
DE = 16
H = 64

NC = 2
NS = 16
NW = NC * NS
CH_E = 128

NCHUNK = 80
EW = NCHUNK * CH_E
EP = EW * NW

RPT = ((N + NS - 1) // NS + 7) // 8 * 8
NP = RPT * NS


def _proj_body(x_ref, wa_ref, wb_ref, xr_ref, xs_ref):
    xb = x_ref[...]
    xr_ref[...] = jnp.dot(xb, wa_ref[...], preferred_element_type=jnp.float32)
    xs_ref[...] = jnp.dot(xb, wb_ref[...], preferred_element_type=jnp.float32)


def _project(x, wa, wb):
    blk = 1000
    grid = N // blk
    return pl.pallas_call(
        _proj_body,
        grid=(grid,),
        in_specs=[
            pl.BlockSpec((blk, D), lambda i: (i, 0)),
            pl.BlockSpec((D, H), lambda i: (0, 0)),
            pl.BlockSpec((D, H), lambda i: (0, 0)),
        ],
        out_specs=[
            pl.BlockSpec((blk, H), lambda i: (i, 0)),
            pl.BlockSpec((blk, H), lambda i: (i, 0)),
        ],
        out_shape=[
            jax.ShapeDtypeStruct((N, H), jnp.float32),
            jax.ShapeDtypeStruct((N, H), jnp.float32),
        ],
    )(x, wa, wb)


def _gather_body(xr_hbm, xs_hbm, idxr_hbm, idxs_hbm, g_hbm,
                 idxr_v, idxs_v, bufr0, bufs0, bufr1, bufs1,
                 semr0, sems0, semr1, sems1):
    c = lax.axis_index("c")
    s = lax.axis_index("s")
    wid = s * NC + c
    ibase = pl.multiple_of(wid * NCHUNK, 8)
    pltpu.sync_copy(idxr_hbm.at[pl.ds(ibase, NCHUNK)], idxr_v)
    pltpu.sync_copy(idxs_hbm.at[pl.ds(ibase, NCHUNK)], idxs_v)
    ebase = wid * EW
    bufsets = ((bufr0, bufs0, semr0, sems0), (bufr1, bufs1, semr1, sems1))

    def issue(j, bset):
        br, bs, sr, ss = bset
        pltpu.async_copy(xr_hbm.at[idxr_v.at[j]], br, sr)
        pltpu.async_copy(xs_hbm.at[idxs_v.at[j]], bs, ss)

    def consume(j, bset, i_loop):
        br, bs, sr, ss = bset
        pltpu.make_async_copy(xr_hbm.at[idxr_v.at[j]], br, sr).wait()
        pltpu.make_async_copy(xs_hbm.at[idxs_v.at[j]], bs, ss).wait()

        @plsc.parallel_loop(0, CH_E, unroll=4)
        def row(i):
            for cc in range(H // 16):
                sl = pl.ds(cc * 16, 16)
                plsc.addupdate(br.at[i, sl], bs[i, sl])

        off = pl.multiple_of(ebase + j * CH_E, 8)
        pltpu.sync_copy(br, g_hbm.at[pl.ds(off, CH_E)])

        @pl.when(i_loop < NCHUNK // 2 - 1)
        def _():
            issue(j + 2, bset)

    issue(0, bufsets[0])
    issue(1, bufsets[1])

    def pair(i, carry):
        consume(2 * i, bufsets[0], i)
        consume(2 * i + 1, bufsets[1], i)
        return carry

    lax.fori_loop(0, NCHUNK // 2, pair, 0)


def _edge_gather(xr, xs, idxr2d, idxs2d):
    mesh = plsc.VectorSubcoreMesh(core_axis_name="c", subcore_axis_name="s")
    f = pl.kernel(
        _gather_body,
        out_type=jax.ShapeDtypeStruct((EP, H), jnp.float32),
        mesh=mesh,
        compiler_params=pltpu.CompilerParams(use_tc_tiling_on_sc=False),
        scratch_types=[
            pltpu.VMEM((NCHUNK, CH_E), jnp.int32),
            pltpu.VMEM((NCHUNK, CH_E), jnp.int32),
            pltpu.VMEM((CH_E, H), jnp.float32),
            pltpu.VMEM((CH_E, H), jnp.float32),
            pltpu.VMEM((CH_E, H), jnp.float32),
            pltpu.VMEM((CH_E, H), jnp.float32),
            pltpu.SemaphoreType.DMA,
            pltpu.SemaphoreType.DMA,
            pltpu.SemaphoreType.DMA,
            pltpu.SemaphoreType.DMA,
        ],
    )
    return f(xr, xs, idxr2d, idxs2d)


def _edge_mlp_body(g_ref, ea_ref, w1c_ref, be1_ref, w2_ref, be2_ref, out_ref):
    ea = ea_ref[...]
    h = g_ref[...] + jnp.dot(ea, w1c_ref[...], preferred_element_type=jnp.float32)
    h = jnp.maximum(h + be1_ref[...], 0.0)
    out_ref[...] = ea + jnp.dot(h, w2_ref[...], preferred_element_type=jnp.float32) + be2_ref[...]


def _edge_mlp(g, ea_pad, w1c, be1, w2, be2):
    blk = 2048
    grid = EP // blk
    return pl.pallas_call(
        _edge_mlp_body,
        grid=(grid,),
        in_specs=[
            pl.BlockSpec((blk, H), lambda i: (i, 0)),
            pl.BlockSpec((blk, DE), lambda i: (i, 0)),
            pl.BlockSpec((DE, H), lambda i: (0, 0)),
            pl.BlockSpec((1, H), lambda i: (0, 0)),
            pl.BlockSpec((H, DE), lambda i: (0, 0)),
            pl.BlockSpec((1, DE), lambda i: (0, 0)),
        ],
        out_specs=pl.BlockSpec((blk, DE), lambda i: (i, 0)),
        out_shape=jax.ShapeDtypeStruct((EP, DE), jnp.float32),
    )(g, ea_pad, w1c, be1, w2, be2)


def _scatter_body(edges_hbm, sidx_hbm, out_hbm, sidx_v, ebuf, zbuf, acc, sem):
    c = lax.axis_index("c")
    s = lax.axis_index("s")
    wid = s * NC + c

    def zrow(i, carry):
        zbuf[i, :] = jnp.zeros((16,), jnp.float32)
        return carry

    lax.fori_loop(0, RPT, zrow, 0)

    rbase = pl.multiple_of(s * RPT, 8)
    pltpu.sync_copy(zbuf, acc.at[pl.ds(rbase, RPT)])
    plsc.subcore_barrier()
    pltpu.sync_copy(sidx_hbm.at[pl.ds(pl.multiple_of(wid * NCHUNK, 8), NCHUNK)], sidx_v)
    ebase = wid * EW

    def chunk(j, carry):
        off = pl.multiple_of(ebase + j * CH_E, 8)
        pltpu.sync_copy(edges_hbm.at[pl.ds(off, CH_E)], ebuf)
        pltpu.sync_copy(ebuf, acc.at[sidx_v.at[j]], add=True)
        return carry

    lax.fori_loop(0, NCHUNK, chunk, 0)
    plsc.subcore_barrier()
    pltpu.sync_copy(acc.at[pl.ds(rbase, RPT)], zbuf)
    pltpu.sync_copy(zbuf, out_hbm.at[c, pl.ds(rbase, RPT)])


def _segment_scatter(edges_new, sidx2d):
    mesh = plsc.VectorSubcoreMesh(core_axis_name="c", subcore_axis_name="s")
    f = pl.kernel(
        _scatter_body,
        out_type=jax.ShapeDtypeStruct((NC, NP, DE), jnp.float32),
        mesh=mesh,
        compiler_params=pltpu.CompilerParams(use_tc_tiling_on_sc=False),
        scratch_types=[
            pltpu.VMEM((NCHUNK, CH_E), jnp.int32),
            pltpu.VMEM((CH_E, DE), jnp.float32),
            pltpu.VMEM((RPT, DE), jnp.float32),
            pltpu.VMEM_SHARED((NP, DE), jnp.float32),
            pltpu.SemaphoreType.DMA,
        ],
    )
    return f(edges_new, sidx2d)


def _node_body(x_ref, p_ref, wa_ref, wb_ref, bn1_ref, w2_ref, bn2_ref, out_ref):
    xb = x_ref[...]
    a = p_ref[0] + p_ref[1]
    h = jnp.dot(xb, wa_ref[...], preferred_element_type=jnp.float32)
    h = h + jnp.dot(a, wb_ref[...], preferred_element_type=jnp.float32)
    h = jnp.maximum(h + bn1_ref[...], 0.0)
    out_ref[...] = xb + jnp.dot(h, w2_ref[...], preferred_element_type=jnp.float32) + bn2_ref[...]


def _node_mlp(x, partials, wa, wb, bn1, w2, bn2):
    blk = 1000
    grid = N // blk
    return pl.pallas_call(
        _node_body,
        grid=(grid,),
        in_specs=[
            pl.BlockSpec((blk, D), lambda i: (i, 0)),
            pl.BlockSpec((NC, blk, DE), lambda i: (0, i, 0)),
            pl.BlockSpec((D, H), lambda i: (0, 0)),
            pl.BlockSpec((DE, H), lambda i: (0, 0)),
            pl.BlockSpec((1, H), lambda i: (0, 0)),
            pl.BlockSpec((H, D), lambda i: (0, 0)),
            pl.BlockSpec((1, D), lambda i: (0, 0)),
        ],
        out_specs=pl.BlockSpec((blk, D), lambda i: (i, 0)),
        out_shape=jax.ShapeDtypeStruct((N, D), jnp.float32),
    )(x, partials, wa, wb, bn1, w2, bn2)


def kernel(x, edge_index, edge_attr, We1, be1, We2, be2, Wn1, bn1, Wn2, bn2):
    senders = edge_index[0].astype(jnp.int32)
    receivers = edge_index[1].astype(jnp.int32)
    pad = EP - E
    gr = jnp.concatenate([receivers, jnp.zeros((pad,), jnp.int32)]).reshape(-1, CH_E)
    gs = jnp.concatenate([senders, jnp.zeros((pad,), jnp.int32)]).reshape(-1, CH_E)
    sr = jnp.concatenate([receivers, jnp.full((pad,), N, jnp.int32)]).reshape(-1, CH_E)
    ea_pad = jnp.concatenate([edge_attr, jnp.zeros((pad, DE), jnp.float32)])

    xr, xs = _project(x, We1[:D], We1[D:2 * D])
    g = _edge_gather(xr, xs, gr, gs)
    edges_new = _edge_mlp(g, ea_pad, We1[2 * D:], be1.reshape(1, H),
                          We2, be2.reshape(1, DE))
    partials = _segment_scatter(edges_new, sr)
    nodes_new = _node_mlp(x, partials, Wn1[:D], Wn1[D:], bn1.reshape(1, H),
                          Wn2, bn2.reshape(1, D))
    return nodes_new


# speedup vs baseline: 3.5645x; 1.2122x over previous
import functools

import jax
import jax.numpy as jnp
from jax import lax
from jax.experimental import pallas as pl
from jax.experimental.pallas import tpu as pltpu
from jax.experimental.pallas import tpu_sc as plsc

N = 10000
E = 320000
D = 128
DE = 16
H = 64

NC = 2
NS = 16
CH_E = 128
NCHUNKS = E // CH_E
EH = E // 2

N0 = 94
N1 = 62
REM = NCHUNKS - NS * (N0 + N1)
IDXROWS = 2560
WIN = 96

RPT = 632
NP = RPT * NS


def _proj_body(x_ref, wa_ref, wb_ref, be1_ref, xr_ref, xs_ref):
    xb = x_ref[...]
    xr_ref[...] = jnp.dot(xb, wa_ref[...], preferred_element_type=jnp.float32) + be1_ref[...]
    xs_ref[...] = jnp.dot(xb, wb_ref[...], preferred_element_type=jnp.float32)


def _project(x, wa, wb, be1):
    blk = 1000
    return pl.pallas_call(
        _proj_body,
        grid=(N // blk,),
        in_specs=[
            pl.BlockSpec((blk, D), lambda i: (i, 0)),
            pl.BlockSpec((D, H), lambda i: (0, 0)),
            pl.BlockSpec((D, H), lambda i: (0, 0)),
            pl.BlockSpec((1, H), lambda i: (0, 0)),
        ],
        out_specs=[
            pl.BlockSpec((blk, H), lambda i: (i, 0)),
            pl.BlockSpec((blk, H), lambda i: (i, 0)),
        ],
        out_shape=[
            jax.ShapeDtypeStruct((N, H), jnp.float32),
            jax.ShapeDtypeStruct((N, H), jnp.float32),
        ],
    )(x, wa, wb, be1)


def _ea_proj_body(ea_lo_ref, ea_hi_ref, w_ref, out_ref):
    w = w_ref[...]
    lo = jnp.dot(ea_lo_ref[...], w, preferred_element_type=jnp.float32)
    hi = jnp.dot(ea_hi_ref[...], w, preferred_element_type=jnp.float32)
    out_ref[...] = jnp.concatenate([lo, hi], axis=1)


def _ea_project(edge_attr, w1c):
    blk = 2000
    nblk = EH // blk
    return pl.pallas_call(
        _ea_proj_body,
        grid=(nblk,),
        in_specs=[
            pl.BlockSpec((blk, DE), lambda i: (i, 0)),
            pl.BlockSpec((blk, DE), lambda i: (i + nblk, 0)),
            pl.BlockSpec((DE, H), lambda i: (0, 0)),
        ],
        out_specs=pl.BlockSpec((blk, 2 * H), lambda i: (i, 0)),
        out_shape=jax.ShapeDtypeStruct((EH, 2 * H), jnp.float32),
    )(edge_attr, edge_attr, w1c)


def _fused_body(xr_hbm, xs_hbm, ea_hbm, ce2_hbm, be2_hbm, idxr_hbm, idxs_hbm,
                ph_hbm, pa_hbm,
                idxr_v, idxs_v,
                bufr0, bufs0, bufr1, bufs1, bufc, bufa, bev,
                zbufh, zbufa,
                acch, acca,
                semr0, sems0, semr1, sems1):
    c = lax.axis_index("c")
    s = lax.axis_index("s")

    pltpu.sync_copy(be2_hbm, bev)

    def zrow(i, carry):
        for cc in range(H // 16):
            zbufh[i, pl.ds(cc * 16, 16)] = jnp.zeros((16,), jnp.float32)
        zbufa[i, :] = jnp.zeros((16,), jnp.float32)
        return carry

    lax.fori_loop(0, CH_E, zrow, 0)
    rbase = pl.multiple_of(s * RPT, 8)
    for piece, rows in ((0, CH_E), (1, CH_E), (2, CH_E), (3, CH_E), (4, RPT - 4 * CH_E)):
        off = pl.multiple_of(rbase + piece * CH_E, 8)
        pltpu.sync_copy(zbufh.at[pl.ds(0, rows)], acch.at[pl.ds(off, rows)])
        pltpu.sync_copy(zbufa.at[pl.ds(0, rows)], acca.at[pl.ds(off, rows)])
    plsc.subcore_barrier()

    extra = jnp.where(s < REM, 1, 0)
    start0 = s * N0 + jnp.minimum(s, REM)
    start1 = NS * N0 + REM + s * N1
    start = jnp.where(c == 0, start0, start1)
    nch = jnp.where(c == 0, N0 + extra, N1)

    pltpu.sync_copy(idxr_hbm.at[pl.ds(start, WIN)], idxr_v)
    pltpu.sync_copy(idxs_hbm.at[pl.ds(start, WIN)], idxs_v)

    bufsets = ((bufr0, bufs0, semr0, sems0), (bufr1, bufs1, semr1, sems1))

    def issue(i, bset):
        br, bs, sr, ss = bset
        pltpu.async_copy(xr_hbm.at[idxr_v.at[i]], br, sr)
        pltpu.async_copy(xs_hbm.at[idxs_v.at[i]], bs, ss)

    def consume(i, bset):
        br, bs, sr, ss = bset
        ch = start + i
        pltpu.make_async_copy(xr_hbm.at[idxr_v.at[i]], br, sr).wait()
        pltpu.make_async_copy(xs_hbm.at[idxs_v.at[i]], bs, ss).wait()
        pltpu.sync_copy(ce2_hbm.at[pl.ds(ch * (CH_E // 2), CH_E // 2)], bufc)
        pltpu.sync_copy(ea_hbm.at[pl.ds(ch * (CH_E // 2), CH_E // 2)],
                        bufa.at[pl.ds(0, CH_E // 2)])
        pltpu.sync_copy(ea_hbm.at[pl.ds(EH + ch * (CH_E // 2), CH_E // 2)],
                        bufa.at[pl.ds(CH_E // 2, CH_E // 2)])
        be2v = bev[...]

        @plsc.parallel_loop(0, CH_E, unroll=4)
        def earow(k):
            bufa[k, :] = bufa[k, :] + be2v

        @plsc.parallel_loop(0, CH_E // 2, unroll=2)
        def row(k):
            for cc in range(H // 16):
                sl = pl.ds(cc * 16, 16)
                lo = br[k, sl] + bs[k, sl] + bufc[k, sl]
                br[k, sl] = jnp.maximum(lo, 0.0)
                hi = br[64 + k, sl] + bs[64 + k, sl] + bufc[k, pl.ds(H + cc * 16, 16)]
                br[64 + k, sl] = jnp.maximum(hi, 0.0)

        pltpu.sync_copy(br, acch.at[idxr_v.at[i]], add=True)
        pltpu.sync_copy(bufa, acca.at[idxr_v.at[i]], add=True)

        @pl.when(i + 2 < nch)
        def _():
            issue(i + 2, bset)

    issue(0, bufsets[0])

    @pl.when(nch > 1)
    def _():
        issue(1, bufsets[1])

    def chunk(i, carry):
        @pl.when(i % 2 == 0)
        def _():
            consume(i, bufsets[0])

        @pl.when(i % 2 == 1)
        def _():
            consume(i, bufsets[1])

        return carry

    lax.fori_loop(0, nch, chunk, 0)
    plsc.subcore_barrier()

    for piece, rows in ((0, CH_E), (1, CH_E), (2, CH_E), (3, CH_E), (4, RPT - 4 * CH_E)):
        off = pl.multiple_of(rbase + piece * CH_E, 8)
        pltpu.sync_copy(acch.at[pl.ds(off, rows)], zbufh.at[pl.ds(0, rows)])
        pltpu.sync_copy(zbufh.at[pl.ds(0, rows)], ph_hbm.at[c, pl.ds(off, rows)])
        pltpu.sync_copy(acca.at[pl.ds(off, rows)], zbufa.at[pl.ds(0, rows)])
        pltpu.sync_copy(zbufa.at[pl.ds(0, rows)], pa_hbm.at[c, pl.ds(off, rows)])


def _fused_edges(xr, xs, edge_attr, ce2, be2, gidxr, gidxs):
    mesh = plsc.VectorSubcoreMesh(core_axis_name="c", subcore_axis_name="s")
    f = pl.kernel(
        _fused_body,
        out_type=[
            jax.ShapeDtypeStruct((NC, NP, H), jnp.float32),
            jax.ShapeDtypeStruct((NC, NP, DE), jnp.float32),
        ],
        mesh=mesh,
        compiler_params=pltpu.CompilerParams(use_tc_tiling_on_sc=False),
        scratch_types=[
            pltpu.VMEM((WIN, CH_E), jnp.int32),
            pltpu.VMEM((WIN, CH_E), jnp.int32),
            pltpu.VMEM((CH_E, H), jnp.float32),
            pltpu.VMEM((CH_E, H), jnp.float32),
            pltpu.VMEM((CH_E, H), jnp.float32),
            pltpu.VMEM((CH_E, H), jnp.float32),
            pltpu.VMEM((CH_E // 2, 2 * H), jnp.float32),
            pltpu.VMEM((CH_E, DE), jnp.float32),
            pltpu.VMEM((DE,), jnp.float32),
            pltpu.VMEM((CH_E, H), jnp.float32),
            pltpu.VMEM((CH_E, DE), jnp.float32),
            pltpu.VMEM_SHARED((NP, H), jnp.float32),
            pltpu.VMEM_SHARED((NP, DE), jnp.float32),
            pltpu.SemaphoreType.DMA,
            pltpu.SemaphoreType.DMA,
            pltpu.SemaphoreType.DMA,
            pltpu.SemaphoreType.DMA,
        ],
    )
    return f(xr, xs, edge_attr, ce2, be2, gidxr, gidxs)


def _node_body(x_ref, ph_ref, pa_ref, we2_ref,
               wa_ref, wb_ref, bn1_ref, w2_ref, bn2_ref, out_ref):
    xb = x_ref[...]
    hsum = ph_ref[0] + ph_ref[1]
    agg = pa_ref[0] + pa_ref[1]
    agg = agg + jnp.dot(hsum, we2_ref[...], preferred_element_type=jnp.float32)
    h = jnp.dot(xb, wa_ref[...], preferred_element_type=jnp.float32)
    h = h + jnp.dot(agg, wb_ref[...], preferred_element_type=jnp.float32)
    h = jnp.maximum(h + bn1_ref[...], 0.0)
    out_ref[...] = xb + jnp.dot(h, w2_ref[...], preferred_element_type=jnp.float32) + bn2_ref[...]


def _node_mlp(x, ph, pa, we2, wa, wb, bn1, w2, bn2):
    blk = 1000
    return pl.pallas_call(
        _node_body,
        grid=(N // blk,),
        in_specs=[
            pl.BlockSpec((blk, D), lambda i: (i, 0)),
            pl.BlockSpec((NC, blk, H), lambda i: (0, i, 0)),
            pl.BlockSpec((NC, blk, DE), lambda i: (0, i, 0)),
            pl.BlockSpec((H, DE), lambda i: (0, 0)),
            pl.BlockSpec((D, H), lambda i: (0, 0)),
            pl.BlockSpec((DE, H), lambda i: (0, 0)),
            pl.BlockSpec((1, H), lambda i: (0, 0)),
            pl.BlockSpec((H, D), lambda i: (0, 0)),
            pl.BlockSpec((1, D), lambda i: (0, 0)),
        ],
        out_specs=pl.BlockSpec((blk, D), lambda i: (i, 0)),
        out_shape=jax.ShapeDtypeStruct((N, D), jnp.float32),
    )(x, ph, pa, we2, wa, wb, bn1, w2, bn2)


def _pack_idx(v):
    lo = v[:EH].reshape(NCHUNKS, CH_E // 2)
    hi = v[EH:].reshape(NCHUNKS, CH_E // 2)
    packed = jnp.concatenate([lo, hi], axis=1)
    pad = jnp.zeros((IDXROWS - NCHUNKS, CH_E), jnp.int32)
    return jnp.concatenate([packed, pad], axis=0)


def kernel(x, edge_index, edge_attr, We1, be1, We2, be2, Wn1, bn1, Wn2, bn2):
    senders = edge_index[0].astype(jnp.int32)
    receivers = edge_index[1].astype(jnp.int32)
    gidxr = _pack_idx(receivers)
    gidxs = _pack_idx(senders)

    xr, xs = _project(x, We1[:D], We1[D:2 * D], be1.reshape(1, H))
    ce2 = _ea_project(edge_attr, We1[2 * D:])
    ph, pa = _fused_edges(xr, xs, edge_attr, ce2, be2, gidxr, gidxs)
    return _node_mlp(x, ph, pa, We2,
                     Wn1[:D], Wn1[D:], bn1.reshape(1, H), Wn2, bn2.reshape(1, D))


# speedup vs baseline: 6.1911x; 1.7368x over previous
import functools

import jax
import jax.numpy as jnp
from jax import lax
from jax.experimental import pallas as pl
from jax.experimental.pallas import tpu as pltpu
from jax.experimental.pallas import tpu_sc as plsc

N = 10000
E = 320000
D = 128
DE = 16
H = 64

NC = 2
NS = 16
CH_E = 128
NCHUNKS = E // CH_E
E8 = E // 8

N0 = 78
REM = NCHUNKS - 2 * NS * N0
WIN = 80
IDXROWS = 2504

RPT = 632
NP = RPT * NS
W80 = H + DE
W96 = H + 2 * DE


def _proj_body(x_ref, wa_ref, wb_ref, be1_ref, xr_ref, xs_ref):
    xb = x_ref[...]
    xr = jnp.dot(xb, wa_ref[...], preferred_element_type=jnp.float32) + be1_ref[...]
    xr_ref[...] = xr.astype(jnp.bfloat16)
    xs_ref[...] = jnp.dot(xb, wb_ref[...], preferred_element_type=jnp.float32).astype(jnp.bfloat16)


def _project(x, wa, wb, be1):
    blk = 1000
    return pl.pallas_call(
        _proj_body,
        grid=(N // blk,),
        in_specs=[
            pl.BlockSpec((blk, D), lambda i: (i, 0)),
            pl.BlockSpec((D, H), lambda i: (0, 0)),
            pl.BlockSpec((D, H), lambda i: (0, 0)),
            pl.BlockSpec((1, H), lambda i: (0, 0)),
        ],
        out_specs=[
            pl.BlockSpec((blk, H), lambda i: (i, 0)),
            pl.BlockSpec((blk, H), lambda i: (i, 0)),
        ],
        out_shape=[
            jax.ShapeDtypeStruct((N, H), jnp.bfloat16),
            jax.ShapeDtypeStruct((N, H), jnp.bfloat16),
        ],
    )(x, wa, wb, be1)


def _ea_proj_body(ea8_ref, w_ref, c_ref):
    m = jnp.dot(ea8_ref[...], w_ref[...], preferred_element_type=jnp.float32)
    for k in range(4):
        c_ref[k] = m[:, 128 * k:128 * (k + 1)]


def _ea_project(ea8, wkron):
    blk = 400
    return pl.pallas_call(
        _ea_proj_body,
        grid=(E8 // blk,),
        in_specs=[
            pl.BlockSpec((blk, D), lambda i: (i, 0)),
            pl.BlockSpec((D, 4 * D), lambda i: (0, 0)),
        ],
        out_specs=pl.BlockSpec((4, blk, D), lambda i: (0, i, 0)),
        out_shape=jax.ShapeDtypeStruct((4, E8, D), jnp.float32),
    )(ea8, wkron)


def _fused_body(xr_hbm, xs_hbm, c_hbm,
                idxr_hbm, idxs_hbm,
                pc_hbm,
                idxr_v, idxs_v,
                bufr0, bufs0, bufc0,
                bufr1, bufs1, bufc1,
                brf, zbufh, zbfh,
                acch,
                semr0, sems0, semc0, semr1, sems1, semc1):
    c = lax.axis_index("c")
    s = lax.axis_index("s")

    def zrow(i, carry):
        for cc in range(H // 16):
            zbufh[i, pl.ds(cc * 16, 16)] = jnp.zeros((16,), jnp.float32)
        return carry

    lax.fori_loop(0, CH_E, zrow, 0)
    rbase = pl.multiple_of(s * RPT, 8)
    for piece, rows in ((0, CH_E), (1, CH_E), (2, CH_E), (3, CH_E), (4, RPT - 4 * CH_E)):
        off = pl.multiple_of(rbase + piece * CH_E, 8)
        pltpu.sync_copy(zbufh.at[pl.ds(0, rows)], acch.at[pl.ds(off, rows)])
    plsc.subcore_barrier()

    extra = jnp.where(s < REM, 1, 0)
    start0 = s * N0 + jnp.minimum(s, REM)
    start1 = NS * N0 + REM + s * N0
    start = jnp.where(c == 0, start0, start1)
    nch = jnp.where(c == 0, N0 + extra, N0)

    pltpu.sync_copy(idxr_hbm.at[pl.ds(start, WIN)], idxr_v)
    pltpu.sync_copy(idxs_hbm.at[pl.ds(start, WIN)], idxs_v)

    bufsets = (
        (bufr0, bufs0, bufc0, semr0, sems0, semc0),
        (bufr1, bufs1, bufc1, semr1, sems1, semc1),
    )

    def lin_copies(i, bset):
        br, bs, bc, sr, ss, sc = bset
        ch = start + i
        off = pl.multiple_of(ch * (CH_E // 8), 8)
        cps = []
        for k in range(4):
            cps.append(pltpu.make_async_copy(
                c_hbm.at[k, pl.ds(off, CH_E // 8)],
                bc.at[pl.ds(k * (CH_E // 8), CH_E // 8)], sc))
        return cps

    def issue(i, bset):
        br, bs, bc, sr, ss, sc = bset
        pltpu.async_copy(xr_hbm.at[idxr_v.at[i]], br, sr)
        pltpu.async_copy(xs_hbm.at[idxs_v.at[i]], bs, ss)
        for cp in lin_copies(i, bset):
            cp.start()

    def consume(i, bset):
        br, bs, bc, sr, ss, sc = bset
        pltpu.make_async_copy(xr_hbm.at[idxr_v.at[i]], br, sr).wait()
        pltpu.make_async_copy(xs_hbm.at[idxs_v.at[i]], bs, ss).wait()
        for cp in lin_copies(i, bset):
            cp.wait()

        himask = jnp.full((16,), -65536, jnp.int32)

        @plsc.parallel_loop(0, CH_E // 8)
        def row(k):
            for m in range(8):
                e = 8 * k + m
                crow = 16 * (m // 2) + k
                base = (m % 2) * 64
                for g in range(2):
                    wr = plsc.bitcast(br[e, pl.ds(32 * g, 32)], jnp.int32)
                    ws = plsc.bitcast(bs[e, pl.ds(32 * g, 32)], jnp.int32)
                    ra = plsc.bitcast(wr << 16, jnp.float32)
                    rb = plsc.bitcast(wr & himask, jnp.float32)
                    sa = plsc.bitcast(ws << 16, jnp.float32)
                    sb = plsc.bitcast(ws & himask, jnp.float32)
                    sla = pl.ds(base + 32 * g, 16)
                    slb = pl.ds(base + 32 * g + 16, 16)
                    va = ra + sa + bc[crow, sla]
                    vb = rb + sb + bc[crow, slb]
                    brf[e, pl.ds(32 * g, 16)] = jnp.maximum(va, 0.0)
                    brf[e, pl.ds(32 * g + 16, 16)] = jnp.maximum(vb, 0.0)

        pltpu.sync_copy(brf, acch.at[idxr_v.at[i]], add=True)

        @pl.when(i + 2 < nch)
        def _():
            issue(i + 2, bset)

    issue(0, bufsets[0])

    @pl.when(nch > 1)
    def _():
        issue(1, bufsets[1])

    def chunk(i, carry):
        @pl.when(i % 2 == 0)
        def _():
            consume(i, bufsets[0])

        @pl.when(i % 2 == 1)
        def _():
            consume(i, bufsets[1])

        return carry

    lax.fori_loop(0, nch, chunk, 0)
    plsc.subcore_barrier()

    for piece, rows in ((0, CH_E), (1, CH_E), (2, CH_E), (3, CH_E), (4, RPT - 4 * CH_E)):
        off = pl.multiple_of(rbase + piece * CH_E, 8)
        pltpu.sync_copy(acch.at[pl.ds(off, rows)], zbufh.at[pl.ds(0, rows)])

        def cvt(i, carry):
            for g in range(2):
                a = zbufh[i, pl.ds(32 * g, 16)]
                b = zbufh[i, pl.ds(32 * g + 16, 16)]
                zbfh[i, pl.ds(32 * g, 32)] = plsc.pack(
                    a, b, format=plsc.PackFormat.INTERLEAVED)
            return carry

        lax.fori_loop(0, rows, cvt, 0)
        pltpu.sync_copy(zbfh.at[pl.ds(0, rows)], pc_hbm.at[c, pl.ds(off, rows)])


def _fused_edges(xr, xs, cproj, gidxr, gidxs):
    mesh = plsc.VectorSubcoreMesh(core_axis_name="c", subcore_axis_name="s")
    f = pl.kernel(
        _fused_body,
        out_type=jax.ShapeDtypeStruct((NC, NP, H), jnp.bfloat16),
        mesh=mesh,
        compiler_params=pltpu.CompilerParams(
            use_tc_tiling_on_sc=False, needs_layout_passes=False),
        scratch_types=[
            pltpu.VMEM((WIN, CH_E), jnp.int32),
            pltpu.VMEM((WIN, CH_E), jnp.int32),
            pltpu.VMEM((CH_E, H), jnp.bfloat16),
            pltpu.VMEM((CH_E, H), jnp.bfloat16),
            pltpu.VMEM((CH_E // 2, D), jnp.float32),
            pltpu.VMEM((CH_E, H), jnp.bfloat16),
            pltpu.VMEM((CH_E, H), jnp.bfloat16),
            pltpu.VMEM((CH_E // 2, D), jnp.float32),
            pltpu.VMEM((CH_E, H), jnp.float32),
            pltpu.VMEM((CH_E, H), jnp.float32),
            pltpu.VMEM((CH_E, H), jnp.bfloat16),
            pltpu.VMEM_SHARED((NP, H), jnp.float32),
            pltpu.SemaphoreType.DMA,
            pltpu.SemaphoreType.DMA,
            pltpu.SemaphoreType.DMA,
            pltpu.SemaphoreType.DMA,
            pltpu.SemaphoreType.DMA,
            pltpu.SemaphoreType.DMA,
        ],
    )
    return f(xr, xs, cproj, gidxr, gidxs)


def _ea_scatter_body(ea8_hbm, be2_hbm, sidx_hbm, pa_hbm,
                    sidx_v, ba, ea_rows, bev, zbufa, acca, sem):
    c = lax.axis_index("c")
    s = lax.axis_index("s")

    pltpu.sync_copy(be2_hbm, bev)

    def zrow(i, carry):
        zbufa[i, :] = jnp.zeros((16,), jnp.float32)
        return carry

    lax.fori_loop(0, CH_E, zrow, 0)
    rbase = pl.multiple_of(s * RPT, 8)
    for piece, rows in ((0, CH_E), (1, CH_E), (2, CH_E), (3, CH_E), (4, RPT - 4 * CH_E)):
        off = pl.multiple_of(rbase + piece * CH_E, 8)
        pltpu.sync_copy(zbufa.at[pl.ds(0, rows)], acca.at[pl.ds(off, rows)])
    plsc.subcore_barrier()

    extra = jnp.where(s < REM, 1, 0)
    start0 = s * N0 + jnp.minimum(s, REM)
    start1 = NS * N0 + REM + s * N0
    start = jnp.where(c == 0, start0, start1)
    nch = jnp.where(c == 0, N0 + extra, N0)

    pltpu.sync_copy(sidx_hbm.at[pl.ds(start, WIN)], sidx_v)
    be2v = bev[...]

    def chunk(i, carry):
        ch = start + i
        off = pl.multiple_of(ch * (CH_E // 8), 8)
        pltpu.sync_copy(ea8_hbm.at[pl.ds(off, CH_E // 8)], ba)

        @plsc.parallel_loop(0, CH_E // 8)
        def row(k):
            for m in range(8):
                ea_rows[8 * k + m, :] = ba[k, pl.ds(16 * m, 16)] + be2v

        pltpu.sync_copy(ea_rows, acca.at[sidx_v.at[i]], add=True)
        return carry

    lax.fori_loop(0, nch, chunk, 0)
    plsc.subcore_barrier()

    for piece, rows in ((0, CH_E), (1, CH_E), (2, CH_E), (3, CH_E), (4, RPT - 4 * CH_E)):
        off = pl.multiple_of(rbase + piece * CH_E, 8)
        pltpu.sync_copy(acca.at[pl.ds(off, rows)], zbufa.at[pl.ds(0, rows)])
        pltpu.sync_copy(zbufa.at[pl.ds(0, rows)], pa_hbm.at[c, pl.ds(off, rows)])


def _ea_scatter(ea8, be2, gidxr):
    mesh = plsc.VectorSubcoreMesh(core_axis_name="c", subcore_axis_name="s")
    f = pl.kernel(
        _ea_scatter_body,
        out_type=jax.ShapeDtypeStruct((NC, NP, DE), jnp.float32),
        mesh=mesh,
        compiler_params=pltpu.CompilerParams(
            use_tc_tiling_on_sc=False, needs_layout_passes=False),
        scratch_types=[
            pltpu.VMEM((WIN, CH_E), jnp.int32),
            pltpu.VMEM((CH_E // 8, D), jnp.float32),
            pltpu.VMEM((CH_E, DE), jnp.float32),
            pltpu.VMEM((DE,), jnp.float32),
            pltpu.VMEM((CH_E, DE), jnp.float32),
            pltpu.VMEM_SHARED((NP, DE), jnp.float32),
            pltpu.SemaphoreType.DMA,
        ],
    )
    return f(ea8, be2, gidxr)


def _node_body(x_ref, pc_ref, pa_ref, wfold_ref,
               wa_ref, wb_ref, bn1_ref, w2_ref, bn2_ref, out_ref):
    xb = x_ref[...]
    hsum = pc_ref[0].astype(jnp.float32) + pc_ref[1].astype(jnp.float32)
    agg = pa_ref[0] + pa_ref[1]
    h = jnp.dot(xb, wa_ref[...], preferred_element_type=jnp.float32)
    h = h + jnp.dot(hsum, wfold_ref[...], preferred_element_type=jnp.float32)
    h = h + jnp.dot(agg, wb_ref[...], preferred_element_type=jnp.float32)
    h = jnp.maximum(h + bn1_ref[...], 0.0)
    out_ref[...] = xb + jnp.dot(h, w2_ref[...], preferred_element_type=jnp.float32) + bn2_ref[...]


def _node_mlp(x, pc, pa, wfold, wa, wb, bn1, w2, bn2):
    blk = 1000
    return pl.pallas_call(
        _node_body,
        grid=(N // blk,),
        in_specs=[
            pl.BlockSpec((blk, D), lambda i: (i, 0)),
            pl.BlockSpec((NC, blk, H), lambda i: (0, i, 0)),
            pl.BlockSpec((NC, blk, DE), lambda i: (0, i, 0)),
            pl.BlockSpec((H, H), lambda i: (0, 0)),
            pl.BlockSpec((D, H), lambda i: (0, 0)),
            pl.BlockSpec((DE, H), lambda i: (0, 0)),
            pl.BlockSpec((1, H), lambda i: (0, 0)),
            pl.BlockSpec((H, D), lambda i: (0, 0)),
            pl.BlockSpec((1, D), lambda i: (0, 0)),
        ],
        out_specs=pl.BlockSpec((blk, D), lambda i: (i, 0)),
        out_shape=jax.ShapeDtypeStruct((N, D), jnp.float32),
    )(x, pc, pa, wfold, wa, wb, bn1, w2, bn2)


def _pad_idx(v):
    m = v.reshape(NCHUNKS, CH_E)
    pad = jnp.zeros((IDXROWS - NCHUNKS, CH_E), jnp.int32)
    return jnp.concatenate([m, pad], axis=0)


def kernel(x, edge_index, edge_attr, We1, be1, We2, be2, Wn1, bn1, Wn2, bn2):
    senders = edge_index[0].astype(jnp.int32)
    receivers = edge_index[1].astype(jnp.int32)
    gidxr = _pad_idx(receivers)
    gidxs = _pad_idx(senders)
    ea8 = jax.lax.optimization_barrier(edge_attr.reshape(E8, D))
    wkron = jnp.kron(jnp.eye(8, dtype=jnp.float32), We1[2 * D:])

    perm = []
    for g in range(2):
        for t in range(16):
            perm.extend([32 * g + t, 32 * g + 16 + t])
    parr = jnp.array(perm, jnp.int32)
    wa = We1[:D][:, parr]
    wb = We1[D:2 * D][:, parr]
    be1p = be1[parr]
    wn1b = Wn1[D:]
    wfold = We2[parr, :] @ wn1b

    xr, xs = _project(x, wa, wb, be1p.reshape(1, H))
    cproj = _ea_project(ea8, wkron)
    pc = _fused_edges(xr, xs, cproj, gidxr, gidxs)
    pa = _ea_scatter(ea8, be2, gidxr)
    return _node_mlp(x, pc, pa, wfold,
                     Wn1[:D], Wn1[D:], bn1.reshape(1, H), Wn2, bn2.reshape(1, D))
